# baseline (speedup 1.0000x reference)
import functools

import jax
import jax.numpy as jnp
from jax import lax
from jax.experimental import pallas as pl
from jax.experimental.pallas import tpu as pltpu
from jax.experimental.pallas import tpu_sc as plsc

NC = 2
NS = 16
NW = NC * NS
LC_ROWS = 7816
OC = 4


@jax.jit
def _lookup(idx2, main_t, lastcol):
    b, s = idx2.shape
    v, dm = main_t.shape
    d = dm + 1
    rows_per_w = b // NW
    chunk = OC * s
    n_steps = rows_per_w // OC
    lc_per_s = 488
    ngrp = chunk // 16
    mesh = plsc.VectorSubcoreMesh(core_axis_name="c", subcore_axis_name="s")

    @functools.partial(
        pl.kernel,
        mesh=mesh,
        out_type=jax.ShapeDtypeStruct((b, s, d), jnp.float32),
        compiler_params=pltpu.CompilerParams(needs_layout_passes=False),
        scratch_types=[
            pltpu.VMEM((2 * OC, s), jnp.int32),
            pltpu.VMEM((chunk,), jnp.int32),
            pltpu.VMEM((OC, s, d), jnp.float32),
            pltpu.VMEM((chunk, 128), jnp.float32),
            pltpu.VMEM_SHARED((LC_ROWS, 128), jnp.float32),
            pltpu.SemaphoreType.DMA,
            pltpu.SemaphoreType.DMA,
        ],
    )
    def k(idx_hbm, table_hbm, lc_hbm, out_hbm, idx2_v, idxhi_v,
          rows_v, slab_v, lc_sh, sem, sem2):
        cid = lax.axis_index("c")
        sid = lax.axis_index("s")
        wid = sid * NC + cid
        row_base = wid * rows_per_w
        iota16 = lax.iota(jnp.int32, 16)
        p_outer = [
            lax.div(iota16 + g * 16, jnp.int32(s)) for g in range(ngrp)
        ]
        p_inner = [
            lax.rem(iota16 + g * 16, jnp.int32(s)) for g in range(ngrp)
        ]
        col_last = jnp.full((16,), d - 1, jnp.int32)

        so = sid * lc_per_s
        pltpu.sync_copy(
            lc_hbm.at[pl.ds(so, lc_per_s)], lc_sh.at[pl.ds(so, lc_per_s)]
        )

        @pl.when(sid == NS - 1)
        def _stage_tail():
            pltpu.sync_copy(
                lc_hbm.at[pl.ds(NS * lc_per_s, LC_ROWS - NS * lc_per_s)],
                lc_sh.at[pl.ds(NS * lc_per_s, LC_ROWS - NS * lc_per_s)],
            )

        plsc.subcore_barrier()

        def step_body(c, carry):
            outer = row_base + c * OC
            par = lax.rem(c, 2) * OC

            @pl.when(lax.rem(c, 2) == 0)
            def _refresh():
                offr = pl.multiple_of(
                    row_base + lax.div(c, 2) * (2 * OC), 2 * OC
                )
                pltpu.sync_copy(idx_hbm.at[pl.ds(offr, 2 * OC)], idx2_v)

            for g in range(ngrp):
                iv = plsc.load_gather(
                    idx2_v, [p_outer[g] + par, p_inner[g]]
                )
                idxhi_v[pl.ds(g * 16, 16)] = lax.shift_right_logical(iv, 7)
            mains = []
            for j in range(OC):
                mains.append(
                    pltpu.async_copy(
                        table_hbm.at[idx2_v.at[par + j]],
                        rows_v.at[j, :, pl.ds(0, d - 1)],
                        sem,
                    )
                )
            pltpu.async_copy(lc_sh.at[idxhi_v], slab_v, sem2).wait()
            for m in mains:
                m.wait()
            for g in range(ngrp):
                iv = plsc.load_gather(
                    idx2_v, [p_outer[g] + par, p_inner[g]]
                )
                lo = lax.bitwise_and(iv, 127)
                rows16 = iota16 + g * 16
                vals = plsc.load_gather(slab_v, [rows16, lo])
                plsc.store_scatter(
                    rows_v, [p_outer[g], p_inner[g], col_last], vals
                )
            for j in range(OC):
                pltpu.sync_copy(rows_v.at[j], out_hbm.at[outer + j])
            return carry

        lax.fori_loop(0, n_steps, step_body, 0, unroll=False)

    return k(idx2, main_t, lastcol)


def kernel(indices, embeddings):
    b, s = indices.shape
    v, d = embeddings.shape
    idx2 = indices.astype(jnp.int32)
    main_t = embeddings[:, : d - 1]
    lastcol = jnp.pad(
        embeddings[:, d - 1], (0, LC_ROWS * 128 - v)
    ).reshape(LC_ROWS, 128)
    return _lookup(idx2, main_t, lastcol)
